# baseline (speedup 1.0000x reference)
import functools

import jax
import jax.numpy as jnp
from jax import lax
from jax.experimental import pallas as pl
from jax.experimental.pallas import tpu as pltpu
import jax.experimental.pallas.tpu_sc as plsc

_CUTOFF = 5.0


def _swish(x):
    return x * jax.nn.sigmoid(x)


def _wf_body(fijT_ref, rij_ref, W1_ref, b1_ref, W2_ref, b2_ref, out_ref):
    h = lax.dot_general(fijT_ref[...], W1_ref[...], (((0,), (0,)), ((), ())),
                        preferred_element_type=jnp.float32)
    h = _swish(h + b1_ref[...])
    wf = jnp.dot(h, W2_ref[...], preferred_element_type=jnp.float32) + b2_ref[...]
    r = rij_ref[...]
    c = jnp.where(r < _CUTOFF, 0.5 * (jnp.cos(r * (jnp.pi / _CUTOFF)) + 1.0), 0.0)
    wf = wf * c[:, None]
    out_ref[...] = jnp.concatenate([wf, jnp.zeros_like(wf)], axis=1)


def _compute_wf(fijT, rij, W1, b1r, W2, b2r, blk):
    e = fijT.shape[1]
    return pl.pallas_call(
        _wf_body,
        grid=(pl.cdiv(e, blk),),
        in_specs=[
            pl.BlockSpec((fijT.shape[0], blk), lambda i: (0, i)),
            pl.BlockSpec((blk,), lambda i: (i,)),
            pl.BlockSpec(W1.shape, lambda i: (0, 0)),
            pl.BlockSpec(b1r.shape, lambda i: (0, 0)),
            pl.BlockSpec(W2.shape, lambda i: (0, 0)),
            pl.BlockSpec(b2r.shape, lambda i: (0, 0)),
        ],
        out_specs=pl.BlockSpec((blk, 128), lambda i: (i, 0)),
        out_shape=jax.ShapeDtypeStruct((e, 128), jnp.float32),
    )(fijT, rij, W1, b1r, W2, b2r)


def _proj_body(xT_ref, W_ref, out0_ref, out1_ref):
    y = lax.dot_general(xT_ref[...], W_ref[...], (((0,), (0,)), ((), ())),
                        preferred_element_type=jnp.float32)
    out0_ref[...] = y[:, :32]
    out1_ref[...] = y[:, 32:]


def _compute_y(featT, W_in2f, blk):
    n = featT.shape[1]
    half = jax.ShapeDtypeStruct((n, 32), jnp.float32)
    return pl.pallas_call(
        _proj_body,
        grid=(pl.cdiv(n, blk),),
        in_specs=[
            pl.BlockSpec((featT.shape[0], blk), lambda i: (0, i)),
            pl.BlockSpec(W_in2f.shape, lambda i: (0, 0)),
        ],
        out_specs=[pl.BlockSpec((blk, 32), lambda i: (i, 0))] * 2,
        out_shape=[half, half],
    )(featT, W_in2f)


def _sc_middle(src_mat, dst_mat, y0, y1, wf128, n_edges):
    n = y0.shape[0]
    n_chunks = n_edges // 128
    cpt = src_mat.shape[0] // 16
    assert cpt % 2 == 0
    oc = 80
    n_oc = n // oc
    oco = 400
    n_oco = n // oco
    assert n % oc == 0 and oc % 8 == 0 and n % oco == 0 and oco % 8 == 0

    mesh = plsc.VectorSubcoreMesh(core_axis_name="c", subcore_axis_name="s")

    @functools.partial(
        pl.kernel,
        out_type=jax.ShapeDtypeStruct((n, 128), jnp.float32),
        mesh=mesh,
        compiler_params=pltpu.CompilerParams(use_tc_tiling_on_sc=False),
        scratch_types=[
            pltpu.VMEM((128,), jnp.int32),
            pltpu.VMEM((128,), jnp.int32),
            pltpu.VMEM((128, 32), jnp.float32),
            pltpu.VMEM((128, 32), jnp.float32),
            pltpu.VMEM((128,), jnp.int32),
            pltpu.VMEM((128,), jnp.int32),
            pltpu.VMEM((128, 32), jnp.float32),
            pltpu.VMEM((128, 32), jnp.float32),
            pltpu.VMEM_SHARED((n, 32), jnp.float32),
            pltpu.SemaphoreType.DMA,
            pltpu.SemaphoreType.DMA,
            pltpu.SemaphoreType.DMA,
            pltpu.SemaphoreType.DMA,
            pltpu.SemaphoreType.DMA,
            pltpu.SemaphoreType.DMA,
            pltpu.SemaphoreType.DMA,
            pltpu.SemaphoreType.DMA,
        ],
    )
    def body(src_hbm, dst_hbm, y0_hbm, y1_hbm, wf_hbm, out_hbm,
             srcA, dstA, wfA, rowsA, srcB, dstB, wfB, rowsB, agg_sp,
             semIA, semWA, semGA, semIB, semWB, semGB, semSA, semSB):
        c = lax.axis_index("c")
        s = lax.axis_index("s")
        base = s * cpt
        zeros16 = jnp.zeros((16,), jnp.float32)
        nk = (n_oc - s + 15) // 16

        def zb(i, carry):
            rowsA[i, pl.ds(0, 16)] = zeros16
            rowsA[i, pl.ds(16, 16)] = zeros16
            return carry

        lax.fori_loop(0, oc, zb, 0)

        def stage_body(i, carry):
            off = pl.multiple_of((s + 16 * i) * oc, 8)
            pltpu.sync_copy(rowsA.at[pl.ds(0, oc)], agg_sp.at[pl.ds(off, oc)])
            return carry

        lax.fori_loop(0, nk, stage_body, 0)
        plsc.subcore_barrier()

        def ok_for(i):
            return jnp.logical_and(i < cpt, base + i < n_chunks)

        def fire_front(i, srcb, dstb, wfb, semI, semW):
            chunk = base + i

            @pl.when(ok_for(i))
            def _():
                pltpu.async_copy(src_hbm.at[chunk], srcb, semI)
                pltpu.async_copy(dst_hbm.at[chunk], dstb, semI)
                e0 = pl.multiple_of(chunk * 128, 8)

                @pl.when(c == 0)
                def _():
                    pltpu.async_copy(
                        wf_hbm.at[pl.ds(e0, 128), pl.ds(0, 32)], wfb, semW)

                @pl.when(c == 1)
                def _():
                    pltpu.async_copy(
                        wf_hbm.at[pl.ds(e0, 128), pl.ds(32, 32)], wfb, semW)

        def fire_gather(i, srcb, dstb, rowsb, semI, semG, semS):
            chunk = base + i

            @pl.when(jnp.logical_and(jnp.asarray(i) >= 2, ok_for(i - 2)))
            def _():
                pltpu.make_async_copy(rowsb, agg_sp.at[dstb], semS).wait()

            @pl.when(ok_for(i))
            def _():
                pltpu.make_async_copy(src_hbm.at[chunk], srcb, semI).wait()
                pltpu.make_async_copy(dst_hbm.at[chunk], dstb, semI).wait()

                @pl.when(c == 0)
                def _():
                    pltpu.async_copy(y0_hbm.at[srcb], rowsb, semG)

                @pl.when(c == 1)
                def _():
                    pltpu.async_copy(y1_hbm.at[srcb], rowsb, semG)

        def process(i, srcb, dstb, wfb, rowsb, semW, semG, semS):
            chunk = base + i

            @pl.when(ok_for(i))
            def _():
                pltpu.make_async_copy(y0_hbm.at[srcb], rowsb, semG).wait()
                e0 = pl.multiple_of(chunk * 128, 8)
                pltpu.make_async_copy(
                    wf_hbm.at[pl.ds(e0, 128), pl.ds(0, 32)], wfb, semW).wait()

                for e in range(128):
                    for k in range(2):
                        sl = pl.ds(k * 16, 16)
                        rowsb[e, sl] = rowsb[e, sl] * wfb[e, sl]

                pltpu.async_copy(rowsb, agg_sp.at[dstb], semS, add=True)

        A = (srcA, dstA, wfA, rowsA, semIA, semWA, semGA, semSA)
        B = (srcB, dstB, wfB, rowsB, semIB, semWB, semGB, semSB)

        def ff(i, t):
            fire_front(i, t[0], t[1], t[2], t[4], t[5])

        def fg(i, t):
            fire_gather(i, t[0], t[1], t[3], t[4], t[6], t[7])

        def pr(i, t):
            process(i, t[0], t[1], t[2], t[3], t[5], t[6], t[7])

        ff(0, A)
        ff(1, B)
        fg(0, A)

        def pair_body(j, carry):
            i0 = 2 * j
            pr(i0, A)
            ff(i0 + 2, A)
            fg(i0 + 1, B)
            pr(i0 + 1, B)
            ff(i0 + 3, B)
            fg(i0 + 2, A)
            return carry

        lax.fori_loop(0, cpt // 2, pair_body, 0)
        @pl.when(ok_for(cpt - 1))
        def _():
            pltpu.make_async_copy(rowsB, agg_sp.at[dstB], semSB).wait()
        plsc.subcore_barrier()

        nko = (n_oco - s + 15) // 16

        def out_body(i, carry):
            off = pl.multiple_of((s + 16 * i) * oco, 8)
            sl = pl.ds(off, oco)

            @pl.when(c == 0)
            def _():
                pltpu.sync_copy(agg_sp.at[sl], out_hbm.at[sl, pl.ds(0, 32)])

            @pl.when(c == 1)
            def _():
                pltpu.sync_copy(agg_sp.at[sl], out_hbm.at[sl, pl.ds(32, 32)])
            return carry

        lax.fori_loop(0, nko, out_body, 0)

    return body(src_mat, dst_mat, y0, y1, wf128)


def _out_body(agg_ref, W_ref, b_ref, out_ref):
    acc = jnp.dot(agg_ref[...][:, :64], W_ref[...],
                  preferred_element_type=jnp.float32)
    out_ref[...] = _swish(acc + b_ref[...])


def _compute_out(agg128, W_f2out, br, blk):
    n = agg128.shape[0]
    return pl.pallas_call(
        _out_body,
        grid=(n // blk,),
        in_specs=[
            pl.BlockSpec((blk, 128), lambda i: (i, 0)),
            pl.BlockSpec(W_f2out.shape, lambda i: (0, 0)),
            pl.BlockSpec(br.shape, lambda i: (0, 0)),
        ],
        out_specs=pl.BlockSpec((blk, 64), lambda i: (i, 0)),
        out_shape=jax.ShapeDtypeStruct((n, 64), jnp.float32),
    )(agg128, W_f2out, br)


def kernel(feat, edge_index, fij, rij, W1, b1, W2, b2, W_in2f, W_f2out, b_f2out):
    e = fij.shape[0]
    grp = 16 * 2 * 128
    e_virt = ((e + grp - 1) // grp) * grp
    pad = e_virt - e

    src_mat = jnp.pad(edge_index[0], (0, pad)).reshape(-1, 128)
    dst_mat = jnp.pad(edge_index[1], (0, pad)).reshape(-1, 128)

    wf128 = _compute_wf(fij.T, rij, W1, b1[None, :], W2,
                        b2[None, :], blk=4096)
    y0, y1 = _compute_y(feat.T, W_in2f, blk=3200)

    agg128 = _sc_middle(src_mat, dst_mat, y0, y1, wf128, e)

    out = _compute_out(agg128, W_f2out, b_f2out[None, :], blk=1000)
    return out
